# baseline (speedup 1.0000x reference)
import jax
import jax.numpy as jnp
from jax import lax
from jax.experimental import pallas as pl
from jax.experimental.pallas import tpu as pltpu
from jax.experimental.pallas import tpu_sc as plsc

NC = 2
NS = 16
L = 16
CH = 128


def _make_hist(n_pad, ncw):
    epw = ncw * CH
    slc = n_pad // NS
    mesh = plsc.VectorSubcoreMesh(core_axis_name="c", subcore_axis_name="s")

    def body(dst2d, ewp, deg_o, deg_s, dstv, ewv, slice_v, ssem):
        c = lax.axis_index("c")
        s = lax.axis_index("s")
        wid = c * NS + s
        zeros16 = jnp.zeros((L,), jnp.float32)

        def zb(i, _):
            slice_v[pl.ds(i * L, L)] = zeros16
            return 0
        lax.fori_loop(0, slc // L, zb, 0)
        pltpu.sync_copy(slice_v, deg_s.at[pl.ds(s * slc, slc)])
        plsc.subcore_barrier()

        pltpu.sync_copy(dst2d.at[pl.ds(wid * ncw, ncw)], dstv)
        pltpu.sync_copy(ewp.at[pl.ds(wid * epw, epw)], ewv)
        for k0 in range(0, ncw, 20):
            descs = [
                pltpu.async_copy(ewv.at[pl.ds(k * CH, CH)],
                                 deg_s.at[dstv.at[k]], ssem, add=True)
                for k in range(k0, min(k0 + 20, ncw))
            ]
            for d in descs:
                d.wait()
        plsc.subcore_barrier()

        pltpu.sync_copy(deg_s.at[pl.ds(s * slc, slc)], slice_v)
        pltpu.sync_copy(slice_v, deg_o.at[pl.ds(c * n_pad + s * slc, slc)])

    return pl.kernel(
        body,
        out_type=jax.ShapeDtypeStruct((NC * n_pad,), jnp.float32),
        mesh=mesh,
        scratch_types=[
            pltpu.VMEM_SHARED((n_pad,), jnp.float32),
            pltpu.VMEM((ncw, CH), jnp.int32),
            pltpu.VMEM((epw,), jnp.float32),
            pltpu.VMEM((slc,), jnp.float32),
            pltpu.SemaphoreType.DMA,
        ],
        compiler_params=pltpu.CompilerParams(needs_layout_passes=False),
    )


def _make_spmm(n, dh, e_pad):
    ept = e_pad // NS
    nck = ept // CH
    assert nck % 3 == 0 and n % 8 == 0
    rpt8 = (n // (NS * 8)) * 8
    rem_last = n - rpt8 * (NS - 1)
    mesh = plsc.VectorSubcoreMesh(core_axis_name="c", subcore_axis_name="s")

    def _chunks(total):
        return [CH] * (total // CH) + ([total % CH] if total % CH else [])

    def body(src2, dstp, ewp, h2d, out, acc_s,
             srcc0, srcc1, srcc2, dstc0, dstc1, dstc2, ewc0, ewc1, ewc2,
             rows0, rows1, rows2,
             gsem0, gsem1, gsem2, ssem0, ssem1, ssem2,
             dsem0, dsem1, dsem2, isem0, isem1, isem2):
        c = lax.axis_index("c")
        s = lax.axis_index("s")
        cn = c * n
        et = s * ept
        e0 = c * e_pad + s * ept
        zeros16 = jnp.zeros((L,), jnp.float32)
        srcc = (srcc0, srcc1, srcc2)
        dstc = (dstc0, dstc1, dstc2)
        ewc = (ewc0, ewc1, ewc2)
        rows = (rows0, rows1, rows2)
        gsem = (gsem0, gsem1, gsem2)
        ssem = (ssem0, ssem1, ssem2)
        dsem = (dsem0, dsem1, dsem2)
        isem = (isem0, isem1, isem2)

        def zb(i, _):
            for j in range(dh // L):
                rows0[i, pl.ds(j * L, L)] = zeros16
            return 0
        lax.fori_loop(0, CH, zb, 0)

        def _acc_copy(base, total, to_acc):
            off = 0
            for sz in _chunks(total):
                a = acc_s.at[pl.ds(base + off, sz)]
                r = rows0.at[pl.ds(0, sz)]
                if to_acc:
                    pltpu.sync_copy(r, a)
                else:
                    pltpu.sync_copy(a, r)
                    pltpu.sync_copy(r, out.at[pl.ds(cn + base + off, sz)])
                off += sz

        @pl.when(s < NS - 1)
        def _():
            _acc_copy(s * rpt8, rpt8, True)

        @pl.when(s == NS - 1)
        def _():
            _acc_copy((NS - 1) * rpt8, rem_last, True)
        plsc.subcore_barrier()

        pltpu.async_copy(src2.at[pl.ds(e0, CH)], srcc[0], isem[0])
        pltpu.async_copy(ewp.at[pl.ds(et, CH)], ewc[0], isem[0])
        pltpu.async_copy(dstp.at[pl.ds(et, CH)], dstc[0], dsem[0])
        pltpu.make_async_copy(src2.at[pl.ds(0, CH)], srcc[0], isem[0]).wait()
        pltpu.make_async_copy(ewp.at[pl.ds(0, CH)], ewc[0], isem[0]).wait()
        pltpu.async_copy(h2d.at[srcc[0]], rows[0], gsem[0])
        pltpu.async_copy(src2.at[pl.ds(e0 + CH, CH)], srcc[1], isem[1])
        pltpu.async_copy(ewp.at[pl.ds(et + CH, CH)], ewc[1], isem[1])

        def step(t, _):
            for q in range(3):
                nb = (q + 1) % 3
                n2 = (q + 2) % 3
                g = t * 3 + q

                @pl.when(g >= 2)
                def _(nb=nb):
                    pltpu.make_async_copy(h2d.at[pl.ds(0, CH)],
                                          rows[nb], ssem[nb]).wait()

                @pl.when(g + 1 < nck)
                def _(nb=nb, g=g):
                    pltpu.async_copy(dstp.at[pl.ds(et + (g + 1) * CH, CH)],
                                     dstc[nb], dsem[nb])

                @pl.when(g + 2 < nck)
                def _(n2=n2, g=g):
                    pltpu.async_copy(src2.at[pl.ds(e0 + (g + 2) * CH, CH)],
                                     srcc[n2], isem[n2])
                    pltpu.async_copy(ewp.at[pl.ds(et + (g + 2) * CH, CH)],
                                     ewc[n2], isem[n2])

                @pl.when(g + 1 < nck)
                def _(nb=nb):
                    pltpu.make_async_copy(src2.at[pl.ds(0, CH)],
                                          srcc[nb], isem[nb]).wait()
                    pltpu.make_async_copy(ewp.at[pl.ds(0, CH)],
                                          ewc[nb], isem[nb]).wait()
                    pltpu.async_copy(h2d.at[srcc[nb]], rows[nb], gsem[nb])

                pltpu.make_async_copy(h2d.at[pl.ds(0, CH)],
                                      rows[q], gsem[q]).wait()

                @plsc.parallel_loop(0, CH, step=1, unroll=4)
                def _(k2, q=q):
                    cc = plsc.load_gather(
                        ewc[q], [jnp.zeros((L,), jnp.int32) + k2])
                    for j in range(dh // L):
                        sl = pl.ds(j * L, L)
                        rows[q][k2, sl] = rows[q][k2, sl] * cc

                pltpu.make_async_copy(dstp.at[pl.ds(0, CH)],
                                      dstc[q], dsem[q]).wait()
                pltpu.async_copy(rows[q], acc_s.at[dstc[q]],
                                 ssem[q], add=True)
            return 0
        lax.fori_loop(0, nck // 3, step, 0)
        for g in (nck - 2, nck - 1):
            pltpu.make_async_copy(h2d.at[pl.ds(0, CH)],
                                  rows[g % 3], ssem[g % 3]).wait()
        plsc.subcore_barrier()

        @pl.when(s < NS - 1)
        def _():
            _acc_copy(s * rpt8, rpt8, False)

        @pl.when(s == NS - 1)
        def _():
            _acc_copy((NS - 1) * rpt8, rem_last, False)

    return pl.kernel(
        body,
        out_type=jax.ShapeDtypeStruct((NC * n, dh), jnp.float32),
        mesh=mesh,
        scratch_types=[
            pltpu.VMEM_SHARED((n, dh), jnp.float32),
        ] + [pltpu.VMEM((CH,), jnp.int32)] * 6
          + [pltpu.VMEM((CH,), jnp.float32)] * 3
          + [pltpu.VMEM((CH, dh), jnp.float32)] * 3
          + [pltpu.SemaphoreType.DMA] * 12,
        compiler_params=pltpu.CompilerParams(needs_layout_passes=False),
    )


def _mm1_body(x_ref, w_ref, da_ref, db_ref, o_ref):
    dinv = lax.rsqrt(da_ref[...] + db_ref[...] + 1.0)
    o_ref[...] = dinv * lax.dot_general(x_ref[...], w_ref[...],
                                        (((1,), (1,)), ((), ())),
                                        preferred_element_type=jnp.float32)


def _mid_body(s0, s1, h0, h1, da, db, b1, g, bb, w2, o_ref):
    dinv = lax.rsqrt(da[...] + db[...] + 1.0)
    z0 = dinv * (s0[...] + h0[...])
    z1 = dinv * (s1[...] + h1[...])
    z = jnp.concatenate([z0, z1], axis=1) + b1[...]
    z = jnp.maximum(z, 0.0)
    mu = jnp.mean(z, axis=1, keepdims=True)
    zc = z - mu
    var = jnp.mean(zc * zc, axis=1, keepdims=True)
    y = zc * lax.rsqrt(var + 1e-5) * g[...] + bb[...]
    o_ref[...] = dinv * lax.dot_general(y, w2[...], (((1,), (1,)), ((), ())),
                                        preferred_element_type=jnp.float32)


def _fin_body(s0, s1, h0, h1, da, db, b2, o_ref):
    dinv = lax.rsqrt(da[...] + db[...] + 1.0)
    z0 = dinv * (s0[...] + h0[...])
    z1 = dinv * (s1[...] + h1[...])
    o_ref[...] = jnp.concatenate([z0, z1], axis=1) + b2[...]


def kernel(x, edge_index, edge_weight, W1, b1, ln_g, ln_b, W2, b2):
    n, d_in = x.shape
    d_hid = W1.shape[0]
    d_out = W2.shape[0]
    dh = d_hid // NC
    e = edge_index.shape[1]

    step = 3 * NC * NS * CH
    e_pad = -(-e // step) * step
    n_pad = -(-n // (NS * L)) * (NS * L)

    src = edge_index[0].astype(jnp.int32)
    dst = edge_index[1].astype(jnp.int32)
    pad = e_pad - e
    srcp = jnp.concatenate([src, jnp.zeros((pad,), jnp.int32)])
    dstp = jnp.concatenate([dst, jnp.zeros((pad,), jnp.int32)])
    ewp = jnp.concatenate([edge_weight.astype(jnp.float32),
                           jnp.zeros((pad,), jnp.float32)])
    src2 = jnp.concatenate([srcp, srcp + n])

    nw = NC * NS
    ncw = -(-(e_pad // (nw * CH)) // 8) * 8
    hpad = nw * ncw * CH - e_pad
    dst_h = jnp.concatenate(
        [dstp.reshape(nw, -1, CH),
         jnp.zeros((nw, hpad // (nw * CH), CH), jnp.int32)], axis=1,
    ).reshape(-1, CH)
    ew_h = jnp.concatenate(
        [ewp.reshape(nw, -1, CH),
         jnp.zeros((nw, hpad // (nw * CH), CH), jnp.float32)], axis=1,
    ).reshape(-1)

    deg2 = _make_hist(n_pad, ncw)(dst_h, ew_h)
    dega = deg2[:n_pad].reshape(n_pad, 1)
    degb = deg2[n_pad:].reshape(n_pad, 1)

    rb = 400
    g = n // rb
    f32 = jnp.float32
    vspec = pl.BlockSpec((rb, 1), lambda c, i: (i, 0))
    bspec = pl.BlockSpec((1, d_hid), lambda c, i: (0, 0))

    h1 = pl.pallas_call(
        _mm1_body,
        grid=(NC, g),
        in_specs=[
            pl.BlockSpec((rb, d_in), lambda c, i: (i, 0)),
            pl.BlockSpec((dh, d_in), lambda c, i: (c, 0)),
            vspec, vspec,
        ],
        out_specs=pl.BlockSpec((rb, dh), lambda c, i: (c * (n // rb) + i, 0)),
        out_shape=jax.ShapeDtypeStruct((NC * n, dh), f32),
    )(x, W1, dega, degb)

    spmm = _make_spmm(n, dh, e_pad)
    scat1 = spmm(src2, dstp, ewp, h1)

    h2 = pl.pallas_call(
        _mid_body,
        grid=(NC, g),
        in_specs=[
            pl.BlockSpec((rb, dh), lambda c, i: (i, 0)),
            pl.BlockSpec((rb, dh), lambda c, i: (n // rb + i, 0)),
            pl.BlockSpec((rb, dh), lambda c, i: (i, 0)),
            pl.BlockSpec((rb, dh), lambda c, i: (n // rb + i, 0)),
            vspec, vspec, bspec, bspec, bspec,
            pl.BlockSpec((dh, d_hid), lambda c, i: (c, 0)),
        ],
        out_specs=pl.BlockSpec((rb, dh), lambda c, i: (c * (n // rb) + i, 0)),
        out_shape=jax.ShapeDtypeStruct((NC * n, dh), f32),
    )(scat1, scat1, h1, h1, dega, degb, b1.reshape(1, -1),
      ln_g.reshape(1, -1), ln_b.reshape(1, -1), W2)

    scat2 = spmm(src2, dstp, ewp, h2)

    out = pl.pallas_call(
        _fin_body,
        grid=(1, g),
        in_specs=[
            pl.BlockSpec((rb, dh), lambda c, i: (i, 0)),
            pl.BlockSpec((rb, dh), lambda c, i: (n // rb + i, 0)),
            pl.BlockSpec((rb, dh), lambda c, i: (i, 0)),
            pl.BlockSpec((rb, dh), lambda c, i: (n // rb + i, 0)),
            vspec, vspec,
            pl.BlockSpec((1, d_out), lambda c, i: (0, 0)),
        ],
        out_specs=pl.BlockSpec((rb, d_out), lambda c, i: (i, 0)),
        out_shape=jax.ShapeDtypeStruct((n, d_out), f32),
    )(scat2, scat2, h2, h2, dega, degb, b2.reshape(1, -1))

    return out


# speedup vs baseline: 1.0486x; 1.0486x over previous
import jax
import jax.numpy as jnp
from jax import lax
from jax.experimental import pallas as pl
from jax.experimental.pallas import tpu as pltpu
from jax.experimental.pallas import tpu_sc as plsc

NC = 2
NS = 16
L = 16
CH = 128


def _make_hist(n_pad, ncw):
    epw = ncw * CH
    slc = n_pad // NS
    mesh = plsc.VectorSubcoreMesh(core_axis_name="c", subcore_axis_name="s")

    def body(dst2d, ewp, deg_o, deg_s, dstv, ewv, slice_v, ssem):
        c = lax.axis_index("c")
        s = lax.axis_index("s")
        wid = c * NS + s
        zeros16 = jnp.zeros((L,), jnp.float32)

        def zb(i, _):
            slice_v[pl.ds(i * L, L)] = zeros16
            return 0
        lax.fori_loop(0, slc // L, zb, 0)
        pltpu.sync_copy(slice_v, deg_s.at[pl.ds(s * slc, slc)])
        plsc.subcore_barrier()

        pltpu.sync_copy(dst2d.at[pl.ds(wid * ncw, ncw)], dstv)
        pltpu.sync_copy(ewp.at[pl.ds(wid * epw, epw)], ewv)
        for k0 in range(0, ncw, 20):
            descs = [
                pltpu.async_copy(ewv.at[pl.ds(k * CH, CH)],
                                 deg_s.at[dstv.at[k]], ssem, add=True)
                for k in range(k0, min(k0 + 20, ncw))
            ]
            for d in descs:
                d.wait()
        plsc.subcore_barrier()

        pltpu.sync_copy(deg_s.at[pl.ds(s * slc, slc)], slice_v)
        pltpu.sync_copy(slice_v, deg_o.at[pl.ds(c * n_pad + s * slc, slc)])

    return pl.kernel(
        body,
        out_type=jax.ShapeDtypeStruct((NC * n_pad,), jnp.float32),
        mesh=mesh,
        scratch_types=[
            pltpu.VMEM_SHARED((n_pad,), jnp.float32),
            pltpu.VMEM((ncw, CH), jnp.int32),
            pltpu.VMEM((epw,), jnp.float32),
            pltpu.VMEM((slc,), jnp.float32),
            pltpu.SemaphoreType.DMA,
        ],
        compiler_params=pltpu.CompilerParams(needs_layout_passes=False),
    )


def _make_spmm(n, dh, e_pad):
    ept = e_pad // NS
    nck = ept // CH
    assert nck % 3 == 0 and n % 8 == 0
    rpt8 = (n // (NS * 8)) * 8
    rem_last = n - rpt8 * (NS - 1)
    mesh = plsc.VectorSubcoreMesh(core_axis_name="c", subcore_axis_name="s")

    def _chunks(total):
        return [CH] * (total // CH) + ([total % CH] if total % CH else [])

    def body(src2, dstp, ewp, h2d, out, acc_s,
             srcc0, srcc1, srcc2, dstc0, dstc1, dstc2, ewc0, ewc1, ewc2,
             rows0, rows1, rows2,
             gsem0, gsem1, gsem2, ssem0, ssem1, ssem2,
             dsem0, dsem1, dsem2, isem0, isem1, isem2):
        c = lax.axis_index("c")
        s = lax.axis_index("s")
        cn = c * n
        et = s * ept
        e0 = c * e_pad + s * ept
        zeros16 = jnp.zeros((L,), jnp.float32)
        srcc = (srcc0, srcc1, srcc2)
        dstc = (dstc0, dstc1, dstc2)
        ewc = (ewc0, ewc1, ewc2)
        rows = (rows0, rows1, rows2)
        gsem = (gsem0, gsem1, gsem2)
        ssem = (ssem0, ssem1, ssem2)
        dsem = (dsem0, dsem1, dsem2)
        isem = (isem0, isem1, isem2)

        def zb(i, _):
            for j in range(dh // L):
                rows0[i, pl.ds(j * L, L)] = zeros16
            return 0
        lax.fori_loop(0, CH, zb, 0)

        def _acc_copy(base, total, to_acc):
            off = 0
            for sz in _chunks(total):
                a = acc_s.at[pl.ds(base + off, sz)]
                r = rows0.at[pl.ds(0, sz)]
                if to_acc:
                    pltpu.sync_copy(r, a)
                else:
                    pltpu.sync_copy(a, r)
                    pltpu.sync_copy(r, out.at[pl.ds(cn + base + off, sz)])
                off += sz

        @pl.when(s < NS - 1)
        def _():
            _acc_copy(s * rpt8, rpt8, True)

        @pl.when(s == NS - 1)
        def _():
            _acc_copy((NS - 1) * rpt8, rem_last, True)
        plsc.subcore_barrier()

        pltpu.async_copy(src2.at[pl.ds(e0, CH)], srcc[0], isem[0])
        pltpu.async_copy(ewp.at[pl.ds(et, CH)], ewc[0], isem[0])
        pltpu.async_copy(dstp.at[pl.ds(et, CH)], dstc[0], dsem[0])
        pltpu.make_async_copy(src2.at[pl.ds(0, CH)], srcc[0], isem[0]).wait()
        pltpu.make_async_copy(ewp.at[pl.ds(0, CH)], ewc[0], isem[0]).wait()
        pltpu.async_copy(h2d.at[srcc[0]], rows[0], gsem[0])
        pltpu.async_copy(src2.at[pl.ds(e0 + CH, CH)], srcc[1], isem[1])
        pltpu.async_copy(ewp.at[pl.ds(et + CH, CH)], ewc[1], isem[1])

        def step(t, _):
            for q in range(3):
                nb = (q + 1) % 3
                n2 = (q + 2) % 3
                g = t * 3 + q

                @pl.when(g >= 2)
                def _(nb=nb):
                    pltpu.make_async_copy(h2d.at[pl.ds(0, CH)],
                                          rows[nb], ssem[nb]).wait()

                @pl.when(g + 1 < nck)
                def _(nb=nb, g=g):
                    pltpu.async_copy(dstp.at[pl.ds(et + (g + 1) * CH, CH)],
                                     dstc[nb], dsem[nb])

                @pl.when(g + 2 < nck)
                def _(n2=n2, g=g):
                    pltpu.async_copy(src2.at[pl.ds(e0 + (g + 2) * CH, CH)],
                                     srcc[n2], isem[n2])
                    pltpu.async_copy(ewp.at[pl.ds(et + (g + 2) * CH, CH)],
                                     ewc[n2], isem[n2])

                @pl.when(g + 1 < nck)
                def _(nb=nb):
                    pltpu.make_async_copy(src2.at[pl.ds(0, CH)],
                                          srcc[nb], isem[nb]).wait()
                    pltpu.make_async_copy(ewp.at[pl.ds(0, CH)],
                                          ewc[nb], isem[nb]).wait()
                    pltpu.async_copy(h2d.at[srcc[nb]], rows[nb], gsem[nb])

                pltpu.make_async_copy(h2d.at[pl.ds(0, CH)],
                                      rows[q], gsem[q]).wait()

                def scale(k2, _, q=q):
                    for u in range(2):
                        e_ = k2 * 2 + u
                        cc = plsc.load_gather(
                            ewc[q], [jnp.zeros((L,), jnp.int32) + e_])
                        for j in range(dh // L):
                            sl = pl.ds(j * L, L)
                            rows[q][e_, sl] = rows[q][e_, sl] * cc
                    return 0
                lax.fori_loop(0, CH // 2, scale, 0)

                pltpu.make_async_copy(dstp.at[pl.ds(0, CH)],
                                      dstc[q], dsem[q]).wait()
                pltpu.async_copy(rows[q], acc_s.at[dstc[q]],
                                 ssem[q], add=True)
            return 0
        lax.fori_loop(0, nck // 3, step, 0)
        for g in (nck - 2, nck - 1):
            pltpu.make_async_copy(h2d.at[pl.ds(0, CH)],
                                  rows[g % 3], ssem[g % 3]).wait()
        plsc.subcore_barrier()

        @pl.when(s < NS - 1)
        def _():
            _acc_copy(s * rpt8, rpt8, False)

        @pl.when(s == NS - 1)
        def _():
            _acc_copy((NS - 1) * rpt8, rem_last, False)

    return pl.kernel(
        body,
        out_type=jax.ShapeDtypeStruct((NC * n, dh), jnp.float32),
        mesh=mesh,
        scratch_types=[
            pltpu.VMEM_SHARED((n, dh), jnp.float32),
        ] + [pltpu.VMEM((CH,), jnp.int32)] * 6
          + [pltpu.VMEM((CH,), jnp.float32)] * 3
          + [pltpu.VMEM((CH, dh), jnp.float32)] * 3
          + [pltpu.SemaphoreType.DMA] * 12,
        compiler_params=pltpu.CompilerParams(needs_layout_passes=False),
    )


def _mm1_body(x_ref, w_ref, da_ref, db_ref, o_ref):
    dinv = lax.rsqrt(da_ref[...] + db_ref[...] + 1.0)
    o_ref[...] = dinv * lax.dot_general(x_ref[...], w_ref[...],
                                        (((1,), (1,)), ((), ())),
                                        preferred_element_type=jnp.float32)


def _mid_body(s0, s1, h0, h1, da, db, b1, g, bb, w2, o_ref):
    dinv = lax.rsqrt(da[...] + db[...] + 1.0)
    z0 = dinv * (s0[...] + h0[...])
    z1 = dinv * (s1[...] + h1[...])
    z = jnp.concatenate([z0, z1], axis=1) + b1[...]
    z = jnp.maximum(z, 0.0)
    mu = jnp.mean(z, axis=1, keepdims=True)
    zc = z - mu
    var = jnp.mean(zc * zc, axis=1, keepdims=True)
    y = zc * lax.rsqrt(var + 1e-5) * g[...] + bb[...]
    o_ref[...] = dinv * lax.dot_general(y, w2[...], (((1,), (1,)), ((), ())),
                                        preferred_element_type=jnp.float32)


def _fin_body(s0, s1, h0, h1, da, db, b2, o_ref):
    dinv = lax.rsqrt(da[...] + db[...] + 1.0)
    z0 = dinv * (s0[...] + h0[...])
    z1 = dinv * (s1[...] + h1[...])
    o_ref[...] = jnp.concatenate([z0, z1], axis=1) + b2[...]


def kernel(x, edge_index, edge_weight, W1, b1, ln_g, ln_b, W2, b2):
    n, d_in = x.shape
    d_hid = W1.shape[0]
    d_out = W2.shape[0]
    dh = d_hid // NC
    e = edge_index.shape[1]

    step = 3 * NC * NS * CH
    e_pad = -(-e // step) * step
    n_pad = -(-n // (NS * L)) * (NS * L)

    src = edge_index[0].astype(jnp.int32)
    dst = edge_index[1].astype(jnp.int32)
    pad = e_pad - e
    srcp = jnp.concatenate([src, jnp.zeros((pad,), jnp.int32)])
    dstp = jnp.concatenate([dst, jnp.zeros((pad,), jnp.int32)])
    ewp = jnp.concatenate([edge_weight.astype(jnp.float32),
                           jnp.zeros((pad,), jnp.float32)])
    src2 = jnp.concatenate([srcp, srcp + n])

    nw = NC * NS
    ncw = -(-(e_pad // (nw * CH)) // 8) * 8
    hpad = nw * ncw * CH - e_pad
    dst_h = jnp.concatenate(
        [dstp.reshape(nw, -1, CH),
         jnp.zeros((nw, hpad // (nw * CH), CH), jnp.int32)], axis=1,
    ).reshape(-1, CH)
    ew_h = jnp.concatenate(
        [ewp.reshape(nw, -1, CH),
         jnp.zeros((nw, hpad // (nw * CH), CH), jnp.float32)], axis=1,
    ).reshape(-1)

    deg2 = _make_hist(n_pad, ncw)(dst_h, ew_h)
    dega = deg2[:n_pad].reshape(n_pad, 1)
    degb = deg2[n_pad:].reshape(n_pad, 1)

    rb = 400
    g = n // rb
    f32 = jnp.float32
    vspec = pl.BlockSpec((rb, 1), lambda c, i: (i, 0))
    bspec = pl.BlockSpec((1, d_hid), lambda c, i: (0, 0))

    h1 = pl.pallas_call(
        _mm1_body,
        grid=(NC, g),
        in_specs=[
            pl.BlockSpec((rb, d_in), lambda c, i: (i, 0)),
            pl.BlockSpec((dh, d_in), lambda c, i: (c, 0)),
            vspec, vspec,
        ],
        out_specs=pl.BlockSpec((rb, dh), lambda c, i: (c * (n // rb) + i, 0)),
        out_shape=jax.ShapeDtypeStruct((NC * n, dh), f32),
    )(x, W1, dega, degb)

    spmm = _make_spmm(n, dh, e_pad)
    scat1 = spmm(src2, dstp, ewp, h1)

    h2 = pl.pallas_call(
        _mid_body,
        grid=(NC, g),
        in_specs=[
            pl.BlockSpec((rb, dh), lambda c, i: (i, 0)),
            pl.BlockSpec((rb, dh), lambda c, i: (n // rb + i, 0)),
            pl.BlockSpec((rb, dh), lambda c, i: (i, 0)),
            pl.BlockSpec((rb, dh), lambda c, i: (n // rb + i, 0)),
            vspec, vspec, bspec, bspec, bspec,
            pl.BlockSpec((dh, d_hid), lambda c, i: (c, 0)),
        ],
        out_specs=pl.BlockSpec((rb, dh), lambda c, i: (c * (n // rb) + i, 0)),
        out_shape=jax.ShapeDtypeStruct((NC * n, dh), f32),
    )(scat1, scat1, h1, h1, dega, degb, b1.reshape(1, -1),
      ln_g.reshape(1, -1), ln_b.reshape(1, -1), W2)

    scat2 = spmm(src2, dstp, ewp, h2)

    out = pl.pallas_call(
        _fin_body,
        grid=(1, g),
        in_specs=[
            pl.BlockSpec((rb, dh), lambda c, i: (i, 0)),
            pl.BlockSpec((rb, dh), lambda c, i: (n // rb + i, 0)),
            pl.BlockSpec((rb, dh), lambda c, i: (i, 0)),
            pl.BlockSpec((rb, dh), lambda c, i: (n // rb + i, 0)),
            vspec, vspec,
            pl.BlockSpec((1, d_out), lambda c, i: (0, 0)),
        ],
        out_specs=pl.BlockSpec((rb, d_out), lambda c, i: (i, 0)),
        out_shape=jax.ShapeDtypeStruct((n, d_out), f32),
    )(scat2, scat2, h2, h2, dega, degb, b2.reshape(1, -1))

    return out


# speedup vs baseline: 2.4343x; 2.3215x over previous
import jax
import jax.numpy as jnp
from jax import lax
from jax.experimental import pallas as pl
from jax.experimental.pallas import tpu as pltpu
from jax.experimental.pallas import tpu_sc as plsc

NC = 2
NS = 16
L = 16
CH = 128


def _make_hist(n_pad, ncw):
    epw = ncw * CH
    slc = n_pad // NS
    mesh = plsc.VectorSubcoreMesh(core_axis_name="c", subcore_axis_name="s")

    def body(dst2d, ewp, deg_o, deg_s, dstv, ewv, slice_v, ssem):
        c = lax.axis_index("c")
        s = lax.axis_index("s")
        wid = c * NS + s
        zeros16 = jnp.zeros((L,), jnp.float32)

        def zb(i, _):
            slice_v[pl.ds(i * L, L)] = zeros16
            return 0
        lax.fori_loop(0, slc // L, zb, 0)
        pltpu.sync_copy(slice_v, deg_s.at[pl.ds(s * slc, slc)])
        plsc.subcore_barrier()

        pltpu.sync_copy(dst2d.at[pl.ds(wid * ncw, ncw)], dstv)
        pltpu.sync_copy(ewp.at[pl.ds(wid * epw, epw)], ewv)
        for k0 in range(0, ncw, 20):
            descs = [
                pltpu.async_copy(ewv.at[pl.ds(k * CH, CH)],
                                 deg_s.at[dstv.at[k]], ssem, add=True)
                for k in range(k0, min(k0 + 20, ncw))
            ]
            for d in descs:
                d.wait()
        plsc.subcore_barrier()

        pltpu.sync_copy(deg_s.at[pl.ds(s * slc, slc)], slice_v)
        pltpu.sync_copy(slice_v, deg_o.at[pl.ds(c * n_pad + s * slc, slc)])

    return pl.kernel(
        body,
        out_type=jax.ShapeDtypeStruct((NC * n_pad,), jnp.float32),
        mesh=mesh,
        scratch_types=[
            pltpu.VMEM_SHARED((n_pad,), jnp.float32),
            pltpu.VMEM((ncw, CH), jnp.int32),
            pltpu.VMEM((epw,), jnp.float32),
            pltpu.VMEM((slc,), jnp.float32),
            pltpu.SemaphoreType.DMA,
        ],
        compiler_params=pltpu.CompilerParams(needs_layout_passes=False),
    )


def _make_spmm(n, dh, e_pad):
    ept = e_pad // NS
    nck = ept // CH
    assert n % 8 == 0
    rpt8 = (n // (NS * 8)) * 8
    rem_last = n - rpt8 * (NS - 1)
    mesh = plsc.VectorSubcoreMesh(core_axis_name="c", subcore_axis_name="s")

    def _chunks(total):
        return [CH] * (total // CH) + ([total % CH] if total % CH else [])

    def body(src2, dstp, ewp, h2d, out, acc_s,
             srcv, dstc0, dstc1, ewc0, ewc1, rows0, rows1,
             gsem0, gsem1, ssem0, ssem1, dsem0, dsem1, esem0, esem1):
        c = lax.axis_index("c")
        s = lax.axis_index("s")
        cn = c * n
        zeros16 = jnp.zeros((L,), jnp.float32)
        rows = (rows0, rows1)
        dstc = (dstc0, dstc1)
        ewc = (ewc0, ewc1)
        gsem = (gsem0, gsem1)
        ssem = (ssem0, ssem1)
        dsem = (dsem0, dsem1)
        esem = (esem0, esem1)

        def zb(i, _):
            for j in range(dh // L):
                rows0[i, pl.ds(j * L, L)] = zeros16
            return 0
        lax.fori_loop(0, CH, zb, 0)

        def _acc_copy(base, total, to_acc):
            off = 0
            for sz in _chunks(total):
                a = acc_s.at[pl.ds(base + off, sz)]
                r = rows0.at[pl.ds(0, sz)]
                if to_acc:
                    pltpu.sync_copy(r, a)
                else:
                    pltpu.sync_copy(a, r)
                    pltpu.sync_copy(r, out.at[pl.ds(cn + base + off, sz)])
                off += sz

        @pl.when(s < NS - 1)
        def _():
            _acc_copy(s * rpt8, rpt8, True)

        @pl.when(s == NS - 1)
        def _():
            _acc_copy((NS - 1) * rpt8, rem_last, True)
        plsc.subcore_barrier()

        eb = c * e_pad + s * ept
        pltpu.sync_copy(src2.at[pl.ds(eb, ept)], srcv)
        et = s * ept

        dd = {0: pltpu.async_copy(dstp.at[pl.ds(et, CH)], dstc[0], dsem[0])}
        ed = {0: pltpu.async_copy(ewp.at[pl.ds(et, CH)], ewc[0], esem[0])}
        gd = {0: pltpu.async_copy(h2d.at[srcv.at[pl.ds(0, CH)]],
                                  rows[0], gsem[0])}
        sd = {}
        for g in range(nck):
            b = g % 2
            if g + 1 < nck:
                if g >= 1:
                    sd[g - 1].wait()
                o1 = et + (g + 1) * CH
                dd[g + 1] = pltpu.async_copy(dstp.at[pl.ds(o1, CH)],
                                             dstc[1 - b], dsem[1 - b])
                ed[g + 1] = pltpu.async_copy(ewp.at[pl.ds(o1, CH)],
                                             ewc[1 - b], esem[1 - b])
                gd[g + 1] = pltpu.async_copy(
                    h2d.at[srcv.at[pl.ds((g + 1) * CH, CH)]],
                    rows[1 - b], gsem[1 - b])
            gd[g].wait()
            ed[g].wait()
            rb_ = rows[b]
            ew_ = ewc[b]

            def scale(k2, _, rb_=rb_, ew_=ew_):
                for u in range(2):
                    e_ = k2 * 2 + u
                    cc = plsc.load_gather(
                        ew_, [jnp.zeros((L,), jnp.int32) + e_])
                    for j in range(dh // L):
                        sl = pl.ds(j * L, L)
                        rb_[e_, sl] = rb_[e_, sl] * cc
                return 0
            lax.fori_loop(0, CH // 2, scale, 0)
            dd[g].wait()
            sd[g] = pltpu.async_copy(rb_, acc_s.at[dstc[b]],
                                     ssem[b], add=True)
        sd[nck - 2].wait()
        sd[nck - 1].wait()
        plsc.subcore_barrier()

        @pl.when(s < NS - 1)
        def _():
            _acc_copy(s * rpt8, rpt8, False)

        @pl.when(s == NS - 1)
        def _():
            _acc_copy((NS - 1) * rpt8, rem_last, False)

    return pl.kernel(
        body,
        out_type=jax.ShapeDtypeStruct((NC * n, dh), jnp.float32),
        mesh=mesh,
        scratch_types=[
            pltpu.VMEM_SHARED((n, dh), jnp.float32),
            pltpu.VMEM((ept,), jnp.int32),
            pltpu.VMEM((CH,), jnp.int32),
            pltpu.VMEM((CH,), jnp.int32),
            pltpu.VMEM((CH,), jnp.float32),
            pltpu.VMEM((CH,), jnp.float32),
            pltpu.VMEM((CH, dh), jnp.float32),
            pltpu.VMEM((CH, dh), jnp.float32),
        ] + [pltpu.SemaphoreType.DMA] * 8,
        compiler_params=pltpu.CompilerParams(needs_layout_passes=False),
    )


def _mm1_body(x_ref, w_ref, da_ref, db_ref, o_ref):
    dinv = lax.rsqrt(da_ref[...] + db_ref[...] + 1.0)
    o_ref[...] = dinv * lax.dot_general(x_ref[...], w_ref[...],
                                        (((1,), (1,)), ((), ())),
                                        preferred_element_type=jnp.float32)


def _mid_body(s0, s1, h0, h1, da, db, b1, g, bb, w2, o_ref):
    dinv = lax.rsqrt(da[...] + db[...] + 1.0)
    z0 = dinv * (s0[...] + h0[...])
    z1 = dinv * (s1[...] + h1[...])
    z = jnp.concatenate([z0, z1], axis=1) + b1[...]
    z = jnp.maximum(z, 0.0)
    mu = jnp.mean(z, axis=1, keepdims=True)
    zc = z - mu
    var = jnp.mean(zc * zc, axis=1, keepdims=True)
    y = zc * lax.rsqrt(var + 1e-5) * g[...] + bb[...]
    o_ref[...] = dinv * lax.dot_general(y, w2[...], (((1,), (1,)), ((), ())),
                                        preferred_element_type=jnp.float32)


def _fin_body(s0, s1, h0, h1, da, db, b2, o_ref):
    dinv = lax.rsqrt(da[...] + db[...] + 1.0)
    z0 = dinv * (s0[...] + h0[...])
    z1 = dinv * (s1[...] + h1[...])
    o_ref[...] = jnp.concatenate([z0, z1], axis=1) + b2[...]


def kernel(x, edge_index, edge_weight, W1, b1, ln_g, ln_b, W2, b2):
    n, d_in = x.shape
    d_hid = W1.shape[0]
    d_out = W2.shape[0]
    dh = d_hid // NC
    e = edge_index.shape[1]

    step = NS * CH
    e_pad = -(-e // step) * step
    n_pad = -(-n // (NS * L)) * (NS * L)

    src = edge_index[0].astype(jnp.int32)
    dst = edge_index[1].astype(jnp.int32)
    pad = e_pad - e
    srcp = jnp.concatenate([src, jnp.zeros((pad,), jnp.int32)])
    dstp = jnp.concatenate([dst, jnp.zeros((pad,), jnp.int32)])
    ewp = jnp.concatenate([edge_weight.astype(jnp.float32),
                           jnp.zeros((pad,), jnp.float32)])
    src2 = jnp.concatenate([srcp, srcp + n])

    nw = NC * NS
    e_pad_h = -(-e // (nw * CH * 8)) * (nw * CH * 8)
    ncw = e_pad_h // (nw * CH)
    dst_h = jnp.concatenate(
        [dst, jnp.zeros((e_pad_h - e,), jnp.int32)]).reshape(-1, CH)
    ew_h = jnp.concatenate(
        [edge_weight.astype(jnp.float32), jnp.zeros((e_pad_h - e,),
                                                    jnp.float32)])

    deg2 = _make_hist(n_pad, ncw)(dst_h, ew_h)
    dega = deg2[:n_pad].reshape(n_pad, 1)
    degb = deg2[n_pad:].reshape(n_pad, 1)

    rb = 400
    g = n // rb
    f32 = jnp.float32
    vspec = pl.BlockSpec((rb, 1), lambda c, i: (i, 0))
    bspec = pl.BlockSpec((1, d_hid), lambda c, i: (0, 0))

    h1 = pl.pallas_call(
        _mm1_body,
        grid=(NC, g),
        in_specs=[
            pl.BlockSpec((rb, d_in), lambda c, i: (i, 0)),
            pl.BlockSpec((dh, d_in), lambda c, i: (c, 0)),
            vspec, vspec,
        ],
        out_specs=pl.BlockSpec((rb, dh), lambda c, i: (c * (n // rb) + i, 0)),
        out_shape=jax.ShapeDtypeStruct((NC * n, dh), f32),
    )(x, W1, dega, degb)

    spmm = _make_spmm(n, dh, e_pad)
    scat1 = spmm(src2, dstp, ewp, h1)

    h2 = pl.pallas_call(
        _mid_body,
        grid=(NC, g),
        in_specs=[
            pl.BlockSpec((rb, dh), lambda c, i: (i, 0)),
            pl.BlockSpec((rb, dh), lambda c, i: (n // rb + i, 0)),
            pl.BlockSpec((rb, dh), lambda c, i: (i, 0)),
            pl.BlockSpec((rb, dh), lambda c, i: (n // rb + i, 0)),
            vspec, vspec, bspec, bspec, bspec,
            pl.BlockSpec((dh, d_hid), lambda c, i: (c, 0)),
        ],
        out_specs=pl.BlockSpec((rb, dh), lambda c, i: (c * (n // rb) + i, 0)),
        out_shape=jax.ShapeDtypeStruct((NC * n, dh), f32),
    )(scat1, scat1, h1, h1, dega, degb, b1.reshape(1, -1),
      ln_g.reshape(1, -1), ln_b.reshape(1, -1), W2)

    scat2 = spmm(src2, dstp, ewp, h2)

    out = pl.pallas_call(
        _fin_body,
        grid=(1, g),
        in_specs=[
            pl.BlockSpec((rb, dh), lambda c, i: (i, 0)),
            pl.BlockSpec((rb, dh), lambda c, i: (n // rb + i, 0)),
            pl.BlockSpec((rb, dh), lambda c, i: (i, 0)),
            pl.BlockSpec((rb, dh), lambda c, i: (n // rb + i, 0)),
            vspec, vspec,
            pl.BlockSpec((1, d_out), lambda c, i: (0, 0)),
        ],
        out_specs=pl.BlockSpec((rb, d_out), lambda c, i: (i, 0)),
        out_shape=jax.ShapeDtypeStruct((n, d_out), f32),
    )(scat2, scat2, h2, h2, dega, degb, b2.reshape(1, -1))

    return out


# speedup vs baseline: 2.6489x; 1.0881x over previous
import jax
import jax.numpy as jnp
from jax import lax
from jax.experimental import pallas as pl
from jax.experimental.pallas import tpu as pltpu
from jax.experimental.pallas import tpu_sc as plsc

NC = 2
NS = 16
L = 16
CH = 128


def _make_hist(n_pad, ncw):
    epw = ncw * CH
    slc = n_pad // NS
    mesh = plsc.VectorSubcoreMesh(core_axis_name="c", subcore_axis_name="s")

    def body(dst2d, ewp, deg_o, deg_s, dstv, ewv, slice_v, ssem):
        c = lax.axis_index("c")
        s = lax.axis_index("s")
        wid = c * NS + s
        zeros16 = jnp.zeros((L,), jnp.float32)

        def zb(i, _):
            slice_v[pl.ds(i * L, L)] = zeros16
            return 0
        lax.fori_loop(0, slc // L, zb, 0)
        pltpu.sync_copy(slice_v, deg_s.at[pl.ds(s * slc, slc)])
        plsc.subcore_barrier()

        pltpu.sync_copy(dst2d.at[pl.ds(wid * ncw, ncw)], dstv)
        pltpu.sync_copy(ewp.at[pl.ds(wid * epw, epw)], ewv)
        for k0 in range(0, ncw, 20):
            descs = [
                pltpu.async_copy(ewv.at[pl.ds(k * CH, CH)],
                                 deg_s.at[dstv.at[k]], ssem, add=True)
                for k in range(k0, min(k0 + 20, ncw))
            ]
            for d in descs:
                d.wait()
        plsc.subcore_barrier()

        pltpu.sync_copy(deg_s.at[pl.ds(s * slc, slc)], slice_v)
        pltpu.sync_copy(slice_v, deg_o.at[pl.ds(c * n_pad + s * slc, slc)])

    return pl.kernel(
        body,
        out_type=jax.ShapeDtypeStruct((NC * n_pad,), jnp.float32),
        mesh=mesh,
        scratch_types=[
            pltpu.VMEM_SHARED((n_pad,), jnp.float32),
            pltpu.VMEM((ncw, CH), jnp.int32),
            pltpu.VMEM((epw,), jnp.float32),
            pltpu.VMEM((slc,), jnp.float32),
            pltpu.SemaphoreType.DMA,
        ],
        compiler_params=pltpu.CompilerParams(needs_layout_passes=False),
    )


def _make_spmm(n, dh, e_pad):
    ept = e_pad // NS
    nck = ept // CH
    assert n % 8 == 0
    rpt8 = (n // (NS * 8)) * 8
    rem_last = n - rpt8 * (NS - 1)
    mesh = plsc.VectorSubcoreMesh(core_axis_name="c", subcore_axis_name="s")

    def _chunks(total):
        return [CH] * (total // CH) + ([total % CH] if total % CH else [])

    def body(src2, dstp, ewp, h2d, out, acc_s,
             sc0, sc1, sc2, dc0, dc1, dc2, dc3, dc4, dc5, ec0, ec1, ec2,
             rw0, rw1, rw2, gs0, gs1, gs2, ss0, ss1, ss2,
             is0, is1, is2, ds0, ds1, ds2):
        c = lax.axis_index("c")
        s = lax.axis_index("s")
        cn = c * n
        zeros16 = jnp.zeros((L,), jnp.float32)
        srcc = (sc0, sc1, sc2)
        dstc = (dc0, dc1, dc2, dc3, dc4, dc5)
        ewc = (ec0, ec1, ec2)
        rows = (rw0, rw1, rw2)
        gsem = (gs0, gs1, gs2)
        ssem = (ss0, ss1, ss2)
        isem = (is0, is1, is2)
        dsem = (ds0, ds1, ds2)
        rows0 = rw0

        def zb(i, _):
            for j in range(dh // L):
                rows0[i, pl.ds(j * L, L)] = zeros16
            return 0
        lax.fori_loop(0, CH, zb, 0)

        def _acc_copy(base, total, to_acc):
            off = 0
            for sz in _chunks(total):
                a = acc_s.at[pl.ds(base + off, sz)]
                r = rows0.at[pl.ds(0, sz)]
                if to_acc:
                    pltpu.sync_copy(r, a)
                else:
                    pltpu.sync_copy(a, r)
                    pltpu.sync_copy(r, out.at[pl.ds(cn + base + off, sz)])
                off += sz

        @pl.when(s < NS - 1)
        def _():
            _acc_copy(s * rpt8, rpt8, True)

        @pl.when(s == NS - 1)
        def _():
            _acc_copy((NS - 1) * rpt8, rem_last, True)
        plsc.subcore_barrier()

        et = s * ept
        e0 = c * e_pad + s * ept

        def cp_idx(g):
            o = et + g * CH
            i = pltpu.async_copy(src2.at[pl.ds(e0 + g * CH, CH)],
                                 srcc[g % 3], isem[g % 3])
            pltpu.async_copy(ewp.at[pl.ds(o, CH)], ewc[g % 3], isem[g % 3])
            d = pltpu.async_copy(dstp.at[pl.ds(o, CH)],
                                 dstc[g % 6], dsem[g % 3])
            return i, d

        def gather(g):
            return pltpu.async_copy(h2d.at[srcc[g % 3]], rows[g % 3],
                                    gsem[g % 3])

        idp = {g: cp_idx(g) for g in range(min(3, nck))}
        gd = {}
        for g in range(min(2, nck)):
            idp[g][0].wait()
            idp[g][0].wait()
            gd[g] = gather(g)
        sd = {}
        for g in range(nck):
            b = g % 3
            gd[g].wait()
            rb_ = rows[b]
            ew_ = ewc[b]

            def scale(k2, _, rb_=rb_, ew_=ew_):
                cc = plsc.load_gather(ew_, [jnp.zeros((L,), jnp.int32) + k2])
                for j in range(dh // L):
                    sl = pl.ds(j * L, L)
                    rb_[k2, sl] = rb_[k2, sl] * cc
                return 0
            lax.fori_loop(0, CH, scale, 0)
            idp[g][1].wait()
            sd[g] = pltpu.async_copy(rb_, acc_s.at[dstc[g % 6]],
                                     ssem[b], add=True)
            if g >= 1:
                sd[g - 1].wait()
            if g + 3 < nck:
                idp[g + 3] = cp_idx(g + 3)
            if g + 2 < nck:
                idp[g + 2][0].wait()
                idp[g + 2][0].wait()
                gd[g + 2] = gather(g + 2)
        sd[nck - 1].wait()
        plsc.subcore_barrier()

        @pl.when(s < NS - 1)
        def _():
            _acc_copy(s * rpt8, rpt8, False)

        @pl.when(s == NS - 1)
        def _():
            _acc_copy((NS - 1) * rpt8, rem_last, False)

    return pl.kernel(
        body,
        out_type=jax.ShapeDtypeStruct((NC * n, dh), jnp.float32),
        mesh=mesh,
        scratch_types=[
            pltpu.VMEM_SHARED((n, dh), jnp.float32),
        ] + [pltpu.VMEM((CH,), jnp.int32)] * 9
          + [pltpu.VMEM((CH,), jnp.float32)] * 3
          + [pltpu.VMEM((CH, dh), jnp.float32)] * 3
          + [pltpu.SemaphoreType.DMA] * 12,
        compiler_params=pltpu.CompilerParams(needs_layout_passes=False),
    )


def _mm1_body(x_ref, w_ref, da_ref, db_ref, o_ref):
    dinv = lax.rsqrt(da_ref[...] + db_ref[...] + 1.0)
    o_ref[...] = dinv * lax.dot_general(x_ref[...], w_ref[...],
                                        (((1,), (1,)), ((), ())),
                                        preferred_element_type=jnp.float32)


def _mid_body(s0, s1, h0, h1, da, db, b1, g, bb, w2, o_ref):
    dinv = lax.rsqrt(da[...] + db[...] + 1.0)
    z0 = dinv * (s0[...] + h0[...])
    z1 = dinv * (s1[...] + h1[...])
    z = jnp.concatenate([z0, z1], axis=1) + b1[...]
    z = jnp.maximum(z, 0.0)
    mu = jnp.mean(z, axis=1, keepdims=True)
    zc = z - mu
    var = jnp.mean(zc * zc, axis=1, keepdims=True)
    y = zc * lax.rsqrt(var + 1e-5) * g[...] + bb[...]
    o_ref[...] = dinv * lax.dot_general(y, w2[...], (((1,), (1,)), ((), ())),
                                        preferred_element_type=jnp.float32)


def _fin_body(s0, s1, h0, h1, da, db, b2, o_ref):
    dinv = lax.rsqrt(da[...] + db[...] + 1.0)
    z0 = dinv * (s0[...] + h0[...])
    z1 = dinv * (s1[...] + h1[...])
    o_ref[...] = jnp.concatenate([z0, z1], axis=1) + b2[...]


def kernel(x, edge_index, edge_weight, W1, b1, ln_g, ln_b, W2, b2):
    n, d_in = x.shape
    d_hid = W1.shape[0]
    d_out = W2.shape[0]
    dh = d_hid // NC
    e = edge_index.shape[1]

    step = NS * CH
    e_pad = -(-e // step) * step
    n_pad = -(-n // (NS * L)) * (NS * L)

    src = edge_index[0].astype(jnp.int32)
    dst = edge_index[1].astype(jnp.int32)
    pad = e_pad - e
    srcp = jnp.concatenate([src, jnp.zeros((pad,), jnp.int32)])
    dstp = jnp.concatenate([dst, jnp.zeros((pad,), jnp.int32)])
    ewp = jnp.concatenate([edge_weight.astype(jnp.float32),
                           jnp.zeros((pad,), jnp.float32)])
    src2 = jnp.concatenate([srcp, srcp + n])

    nw = NC * NS
    e_pad_h = -(-e // (nw * CH * 8)) * (nw * CH * 8)
    ncw = e_pad_h // (nw * CH)
    dst_h = jnp.concatenate(
        [dst, jnp.zeros((e_pad_h - e,), jnp.int32)]).reshape(-1, CH)
    ew_h = jnp.concatenate(
        [edge_weight.astype(jnp.float32), jnp.zeros((e_pad_h - e,),
                                                    jnp.float32)])

    deg2 = _make_hist(n_pad, ncw)(dst_h, ew_h)
    dega = deg2[:n_pad].reshape(n_pad, 1)
    degb = deg2[n_pad:].reshape(n_pad, 1)

    rb = 400
    g = n // rb
    f32 = jnp.float32
    vspec = pl.BlockSpec((rb, 1), lambda c, i: (i, 0))
    bspec = pl.BlockSpec((1, d_hid), lambda c, i: (0, 0))

    h1 = pl.pallas_call(
        _mm1_body,
        grid=(NC, g),
        in_specs=[
            pl.BlockSpec((rb, d_in), lambda c, i: (i, 0)),
            pl.BlockSpec((dh, d_in), lambda c, i: (c, 0)),
            vspec, vspec,
        ],
        out_specs=pl.BlockSpec((rb, dh), lambda c, i: (c * (n // rb) + i, 0)),
        out_shape=jax.ShapeDtypeStruct((NC * n, dh), f32),
    )(x, W1, dega, degb)

    spmm = _make_spmm(n, dh, e_pad)
    scat1 = spmm(src2, dstp, ewp, h1)

    h2 = pl.pallas_call(
        _mid_body,
        grid=(NC, g),
        in_specs=[
            pl.BlockSpec((rb, dh), lambda c, i: (i, 0)),
            pl.BlockSpec((rb, dh), lambda c, i: (n // rb + i, 0)),
            pl.BlockSpec((rb, dh), lambda c, i: (i, 0)),
            pl.BlockSpec((rb, dh), lambda c, i: (n // rb + i, 0)),
            vspec, vspec, bspec, bspec, bspec,
            pl.BlockSpec((dh, d_hid), lambda c, i: (c, 0)),
        ],
        out_specs=pl.BlockSpec((rb, dh), lambda c, i: (c * (n // rb) + i, 0)),
        out_shape=jax.ShapeDtypeStruct((NC * n, dh), f32),
    )(scat1, scat1, h1, h1, dega, degb, b1.reshape(1, -1),
      ln_g.reshape(1, -1), ln_b.reshape(1, -1), W2)

    scat2 = spmm(src2, dstp, ewp, h2)

    out = pl.pallas_call(
        _fin_body,
        grid=(1, g),
        in_specs=[
            pl.BlockSpec((rb, dh), lambda c, i: (i, 0)),
            pl.BlockSpec((rb, dh), lambda c, i: (n // rb + i, 0)),
            pl.BlockSpec((rb, dh), lambda c, i: (i, 0)),
            pl.BlockSpec((rb, dh), lambda c, i: (n // rb + i, 0)),
            vspec, vspec,
            pl.BlockSpec((1, d_out), lambda c, i: (0, 0)),
        ],
        out_specs=pl.BlockSpec((rb, d_out), lambda c, i: (i, 0)),
        out_shape=jax.ShapeDtypeStruct((n, d_out), f32),
    )(scat2, scat2, h2, h2, dega, degb, b2.reshape(1, -1))

    return out


# speedup vs baseline: 2.6630x; 1.0053x over previous
import jax
import jax.numpy as jnp
from jax import lax
from jax.experimental import pallas as pl
from jax.experimental.pallas import tpu as pltpu
from jax.experimental.pallas import tpu_sc as plsc

NC = 2
NS = 16
L = 16
CH = 128


def _make_hist(n_pad, ncw):
    epw = ncw * CH
    slc = n_pad // NS
    mesh = plsc.VectorSubcoreMesh(core_axis_name="c", subcore_axis_name="s")

    def body(dst2d, ewp, deg_o, deg_s, dstv, ewv, slice_v, ssem):
        c = lax.axis_index("c")
        s = lax.axis_index("s")
        wid = c * NS + s
        zeros16 = jnp.zeros((L,), jnp.float32)

        def zb(i, _):
            slice_v[pl.ds(i * L, L)] = zeros16
            return 0
        lax.fori_loop(0, slc // L, zb, 0)
        pltpu.sync_copy(slice_v, deg_s.at[pl.ds(s * slc, slc)])
        plsc.subcore_barrier()

        pltpu.sync_copy(dst2d.at[pl.ds(wid * ncw, ncw)], dstv)
        pltpu.sync_copy(ewp.at[pl.ds(wid * epw, epw)], ewv)
        for k0 in range(0, ncw, 20):
            descs = [
                pltpu.async_copy(ewv.at[pl.ds(k * CH, CH)],
                                 deg_s.at[dstv.at[k]], ssem, add=True)
                for k in range(k0, min(k0 + 20, ncw))
            ]
            for d in descs:
                d.wait()
        plsc.subcore_barrier()

        pltpu.sync_copy(deg_s.at[pl.ds(s * slc, slc)], slice_v)
        pltpu.sync_copy(slice_v, deg_o.at[pl.ds(c * n_pad + s * slc, slc)])

    return pl.kernel(
        body,
        out_type=jax.ShapeDtypeStruct((NC * n_pad,), jnp.float32),
        mesh=mesh,
        scratch_types=[
            pltpu.VMEM_SHARED((n_pad,), jnp.float32),
            pltpu.VMEM((ncw, CH), jnp.int32),
            pltpu.VMEM((epw,), jnp.float32),
            pltpu.VMEM((slc,), jnp.float32),
            pltpu.SemaphoreType.DMA,
        ],
        compiler_params=pltpu.CompilerParams(needs_layout_passes=False),
    )


def _make_spmm(n, dh, e_pad):
    ept = e_pad // NS
    nck = ept // CH
    assert n % 8 == 0
    rpt8 = (n // (NS * 8)) * 8
    rem_last = n - rpt8 * (NS - 1)
    mesh = plsc.VectorSubcoreMesh(core_axis_name="c", subcore_axis_name="s")

    def _chunks(total):
        return [CH] * (total // CH) + ([total % CH] if total % CH else [])

    def body(src2, dstp, ewp, h2d, out, acc_s,
             sc0, sc1, sc2, dc0, dc1, dc2, dc3, dc4, dc5, ec0, ec1, ec2,
             rw0, rw1, rw2, gs0, gs1, gs2, ss0, ss1, ss2,
             is0, is1, is2, ds0, ds1, ds2):
        c = lax.axis_index("c")
        s = lax.axis_index("s")
        cn = c * n
        zeros16 = jnp.zeros((L,), jnp.float32)
        srcc = (sc0, sc1, sc2)
        dstc = (dc0, dc1, dc2, dc3, dc4, dc5)
        ewc = (ec0, ec1, ec2)
        rows = (rw0, rw1, rw2)
        gsem = (gs0, gs1, gs2)
        ssem = (ss0, ss1, ss2)
        isem = (is0, is1, is2)
        dsem = (ds0, ds1, ds2)
        rows0 = rw0

        def zb(i, _):
            for j in range(dh // L):
                rows0[i, pl.ds(j * L, L)] = zeros16
            return 0
        lax.fori_loop(0, CH, zb, 0)

        def _acc_copy(base, total, to_acc):
            off = 0
            for sz in _chunks(total):
                a = acc_s.at[pl.ds(base + off, sz)]
                r = rows0.at[pl.ds(0, sz)]
                if to_acc:
                    pltpu.sync_copy(r, a)
                else:
                    pltpu.sync_copy(a, r)
                    pltpu.sync_copy(r, out.at[pl.ds(cn + base + off, sz)])
                off += sz

        @pl.when(s < NS - 1)
        def _():
            _acc_copy(s * rpt8, rpt8, True)

        @pl.when(s == NS - 1)
        def _():
            _acc_copy((NS - 1) * rpt8, rem_last, True)
        plsc.subcore_barrier()

        et = s * ept
        e0 = c * e_pad + s * ept

        def cp_idx(g):
            o = et + g * CH
            i = pltpu.async_copy(src2.at[pl.ds(e0 + g * CH, CH)],
                                 srcc[g % 3], isem[g % 3])
            pltpu.async_copy(ewp.at[pl.ds(o, CH)], ewc[g % 3], isem[g % 3])
            d = pltpu.async_copy(dstp.at[pl.ds(o, CH)],
                                 dstc[g % 6], dsem[g % 3])
            return i, d

        def gather(g):
            return pltpu.async_copy(h2d.at[srcc[g % 3]], rows[g % 3],
                                    gsem[g % 3])

        idp = {g: cp_idx(g) for g in range(min(3, nck))}
        gd = {}
        for g in range(min(2, nck)):
            idp[g][0].wait()
            idp[g][0].wait()
            gd[g] = gather(g)
        sd = {}
        for g in range(nck):
            b = g % 3
            gd[g].wait()
            rb_ = rows[b]
            ew_ = ewc[b]

            def scale(k2, _, rb_=rb_, ew_=ew_):
                for u in range(2):
                    e_ = k2 * 2 + u
                    cc = plsc.load_gather(
                        ew_, [jnp.zeros((L,), jnp.int32) + e_])
                    for j in range(dh // L):
                        sl = pl.ds(j * L, L)
                        rb_[e_, sl] = rb_[e_, sl] * cc
                return 0
            lax.fori_loop(0, CH // 2, scale, 0)
            idp[g][1].wait()
            sd[g] = pltpu.async_copy(rb_, acc_s.at[dstc[g % 6]],
                                     ssem[b], add=True)
            if g >= 1:
                sd[g - 1].wait()
            if g + 3 < nck:
                idp[g + 3] = cp_idx(g + 3)
            if g + 2 < nck:
                idp[g + 2][0].wait()
                idp[g + 2][0].wait()
                gd[g + 2] = gather(g + 2)
        sd[nck - 1].wait()
        plsc.subcore_barrier()

        @pl.when(s < NS - 1)
        def _():
            _acc_copy(s * rpt8, rpt8, False)

        @pl.when(s == NS - 1)
        def _():
            _acc_copy((NS - 1) * rpt8, rem_last, False)

    return pl.kernel(
        body,
        out_type=jax.ShapeDtypeStruct((NC * n, dh), jnp.float32),
        mesh=mesh,
        scratch_types=[
            pltpu.VMEM_SHARED((n, dh), jnp.float32),
        ] + [pltpu.VMEM((CH,), jnp.int32)] * 9
          + [pltpu.VMEM((CH,), jnp.float32)] * 3
          + [pltpu.VMEM((CH, dh), jnp.float32)] * 3
          + [pltpu.SemaphoreType.DMA] * 12,
        compiler_params=pltpu.CompilerParams(needs_layout_passes=False),
    )


def _mm1_body(x_ref, w_ref, da_ref, db_ref, o_ref):
    dinv = lax.rsqrt(da_ref[...] + db_ref[...] + 1.0)
    o_ref[...] = dinv * lax.dot_general(x_ref[...], w_ref[...],
                                        (((1,), (1,)), ((), ())),
                                        preferred_element_type=jnp.float32)


def _mid_body(s0, s1, h0, h1, da, db, b1, g, bb, w2, o_ref):
    dinv = lax.rsqrt(da[...] + db[...] + 1.0)
    z0 = dinv * (s0[...] + h0[...])
    z1 = dinv * (s1[...] + h1[...])
    z = jnp.concatenate([z0, z1], axis=1) + b1[...]
    z = jnp.maximum(z, 0.0)
    mu = jnp.mean(z, axis=1, keepdims=True)
    zc = z - mu
    var = jnp.mean(zc * zc, axis=1, keepdims=True)
    y = zc * lax.rsqrt(var + 1e-5) * g[...] + bb[...]
    o_ref[...] = dinv * lax.dot_general(y, w2[...], (((1,), (1,)), ((), ())),
                                        preferred_element_type=jnp.float32)


def _fin_body(s0, s1, h0, h1, da, db, b2, o_ref):
    dinv = lax.rsqrt(da[...] + db[...] + 1.0)
    z0 = dinv * (s0[...] + h0[...])
    z1 = dinv * (s1[...] + h1[...])
    o_ref[...] = jnp.concatenate([z0, z1], axis=1) + b2[...]


def kernel(x, edge_index, edge_weight, W1, b1, ln_g, ln_b, W2, b2):
    n, d_in = x.shape
    d_hid = W1.shape[0]
    d_out = W2.shape[0]
    dh = d_hid // NC
    e = edge_index.shape[1]

    step = NS * CH
    e_pad = -(-e // step) * step
    n_pad = -(-n // (NS * L)) * (NS * L)

    src = edge_index[0].astype(jnp.int32)
    dst = edge_index[1].astype(jnp.int32)
    pad = e_pad - e
    srcp = jnp.concatenate([src, jnp.zeros((pad,), jnp.int32)])
    dstp = jnp.concatenate([dst, jnp.zeros((pad,), jnp.int32)])
    ewp = jnp.concatenate([edge_weight.astype(jnp.float32),
                           jnp.zeros((pad,), jnp.float32)])
    src2 = jnp.concatenate([srcp, srcp + n])

    nw = NC * NS
    e_pad_h = -(-e // (nw * CH * 8)) * (nw * CH * 8)
    ncw = e_pad_h // (nw * CH)
    dst_h = jnp.concatenate(
        [dst, jnp.zeros((e_pad_h - e,), jnp.int32)]).reshape(-1, CH)
    ew_h = jnp.concatenate(
        [edge_weight.astype(jnp.float32), jnp.zeros((e_pad_h - e,),
                                                    jnp.float32)])

    deg2 = _make_hist(n_pad, ncw)(dst_h, ew_h)
    dega = deg2[:n_pad].reshape(n_pad, 1)
    degb = deg2[n_pad:].reshape(n_pad, 1)

    rb = 400
    g = n // rb
    f32 = jnp.float32
    vspec = pl.BlockSpec((rb, 1), lambda c, i: (i, 0))
    bspec = pl.BlockSpec((1, d_hid), lambda c, i: (0, 0))

    h1 = pl.pallas_call(
        _mm1_body,
        grid=(NC, g),
        in_specs=[
            pl.BlockSpec((rb, d_in), lambda c, i: (i, 0)),
            pl.BlockSpec((dh, d_in), lambda c, i: (c, 0)),
            vspec, vspec,
        ],
        out_specs=pl.BlockSpec((rb, dh), lambda c, i: (c * (n // rb) + i, 0)),
        out_shape=jax.ShapeDtypeStruct((NC * n, dh), f32),
    )(x, W1, dega, degb)

    spmm = _make_spmm(n, dh, e_pad)
    scat1 = spmm(src2, dstp, ewp, h1)

    h2 = pl.pallas_call(
        _mid_body,
        grid=(NC, g),
        in_specs=[
            pl.BlockSpec((rb, dh), lambda c, i: (i, 0)),
            pl.BlockSpec((rb, dh), lambda c, i: (n // rb + i, 0)),
            pl.BlockSpec((rb, dh), lambda c, i: (i, 0)),
            pl.BlockSpec((rb, dh), lambda c, i: (n // rb + i, 0)),
            vspec, vspec, bspec, bspec, bspec,
            pl.BlockSpec((dh, d_hid), lambda c, i: (c, 0)),
        ],
        out_specs=pl.BlockSpec((rb, dh), lambda c, i: (c * (n // rb) + i, 0)),
        out_shape=jax.ShapeDtypeStruct((NC * n, dh), f32),
    )(scat1, scat1, h1, h1, dega, degb, b1.reshape(1, -1),
      ln_g.reshape(1, -1), ln_b.reshape(1, -1), W2)

    scat2 = spmm(src2, dstp, ewp, h2)

    out = pl.pallas_call(
        _fin_body,
        grid=(1, g),
        in_specs=[
            pl.BlockSpec((rb, dh), lambda c, i: (i, 0)),
            pl.BlockSpec((rb, dh), lambda c, i: (n // rb + i, 0)),
            pl.BlockSpec((rb, dh), lambda c, i: (i, 0)),
            pl.BlockSpec((rb, dh), lambda c, i: (n // rb + i, 0)),
            vspec, vspec,
            pl.BlockSpec((1, d_out), lambda c, i: (0, 0)),
        ],
        out_specs=pl.BlockSpec((rb, d_out), lambda c, i: (i, 0)),
        out_shape=jax.ShapeDtypeStruct((n, d_out), f32),
    )(scat2, scat2, h2, h2, dega, degb, b2.reshape(1, -1))

    return out
